# Initial kernel scaffold; baseline (speedup 1.0000x reference)
import jax
import jax.numpy as jnp
from jax.experimental import pallas as pl


def kernel(x, x_paths_3, x_paths_5, row_3, col_3, row_5, col_5, Wa_3, ba_3, Wp_3, bp_3, convw_3, convb_3, Wa_5, ba_5, Wp_5, bp_5, convw_5, convb_5):
    raise NotImplementedError("write your pallas kernel here")


# speedup vs baseline: 1.4182x; 1.4182x over previous
import functools

import jax
import jax.numpy as jnp
from jax import lax
from jax.experimental import pallas as pl
from jax.experimental.pallas import tpu as pltpu
from jax.experimental.pallas import tpu_sc as plsc

C = 128
DEPTH = 2
NS = 16
NCORE = 2
CHUNK = 2048
CHUNK_PAD = CHUNK + 16
ZB = 128
NBUF = 3
CNTW = 8


def _sc_scatter_sums(src, gidx, bidx, nbins):
  L = gidx.shape[0]
  nchunks = -(-nbins // CHUNK)
  nbins_pad = nchunks * CHUNK
  EPT = L // NS
  assert L % NS == 0
  NV = -(-EPT // 16)
  CAP = EPT + 16
  RPT = CHUNK // NS

  mesh = plsc.VectorSubcoreMesh(core_axis_name="c", subcore_axis_name="s",
                                num_cores=NCORE, num_subcores=NS)

  @functools.partial(
      pl.kernel,
      out_type=(
          jax.ShapeDtypeStruct((nbins_pad, C), jnp.float32),
          jax.ShapeDtypeStruct((nbins_pad,), jnp.float32),
      ),
      mesh=mesh,
      compiler_params=pltpu.CompilerParams(needs_layout_passes=False),
      scratch_types=(
          pltpu.VMEM((NV * 16,), jnp.int32),
          pltpu.VMEM((NV * 16,), jnp.int32),
          pltpu.VMEM((CAP,), jnp.int32),
          pltpu.VMEM((CAP,), jnp.int32),
          pltpu.VMEM((NBUF * 16, C), jnp.float32),
          pltpu.VMEM((ZB, C), jnp.float32),
          pltpu.VMEM((ZB,), jnp.float32),
          pltpu.VMEM((16,), jnp.float32),
          pltpu.VMEM_SHARED((CHUNK_PAD, C), jnp.float32),
          pltpu.VMEM_SHARED((CHUNK_PAD,), jnp.float32),
          pltpu.SemaphoreType.DMA((NBUF,)),
      ),
  )
  def sck(src_r, gidx_r, bidx_r, ones_r, zc_r, z8_r, sum_r, cnt_r,
          col_v, row_v, cidx_v, lcol_v, gbuf, zbuf, zcnt, ones_v,
          acc, accn, gsem):
    core = lax.axis_index("c")
    sub = lax.axis_index("s")
    ebase = sub * EPT
    pltpu.sync_copy(bidx_r.at[pl.ds(ebase, EPT)], col_v.at[pl.ds(0, EPT)])
    pltpu.sync_copy(gidx_r.at[pl.ds(ebase, EPT)], row_v.at[pl.ds(0, EPT)])
    pltpu.sync_copy(ones_r, ones_v)
    pltpu.sync_copy(zc_r, zbuf)
    pltpu.sync_copy(z8_r, zcnt)
    lane = lax.broadcasted_iota(jnp.int32, (16,), 0)

    def chunk_body(chunk):
      lo = chunk * CHUNK
      for z in range(RPT // ZB):
        r0 = sub * RPT + z * ZB
        pltpu.sync_copy(zbuf, acc.at[pl.ds(r0, ZB)])
        pltpu.sync_copy(zcnt, accn.at[pl.ds(r0, ZB)])
      plsc.subcore_barrier()

      def scan_body(v, wptr):
        cv = col_v[pl.ds(v * 16, 16)]
        rv = row_v[pl.ds(v * 16, 16)]
        t = cv - lo
        m = (t >= 0) & (t < CHUNK) & (v * 16 + lane < EPT)
        mi = jnp.where(m, 1, 0)
        pos = wptr + plsc.cumsum(mi) - 1
        plsc.store_scatter(cidx_v, [pos], rv, mask=m)
        plsc.store_scatter(lcol_v, [pos], t, mask=m)
        total = lax.squeeze(lax.slice(pos, (15,), (16,)), (0,)) + 1 - wptr
        return wptr + total

      n = lax.fori_loop(0, NV, scan_body, jnp.int32(0), unroll=2)
      plsc.store_scatter(cidx_v, [n + lane], jnp.zeros((16,), jnp.int32))
      plsc.store_scatter(lcol_v, [n + lane], jnp.full((16,), CHUNK, jnp.int32))
      ngroups = (n + 15) // 16

      def fire(j):
        slot = lax.rem(j, NBUF)
        idxv = cidx_v[pl.ds(j * 16, 16)]
        pltpu.async_copy(src_r.at[idxv], gbuf.at[pl.ds(slot * 16, 16)],
                         gsem.at[slot])

      for b in range(NBUF):
        @pl.when(b < ngroups)
        def _():
          fire(jnp.int32(b))

      def step(j, carry):
        slot = lax.rem(j, NBUF)
        idxv = cidx_v[pl.ds(j * 16, 16)]
        pltpu.make_async_copy(src_r.at[idxv],
                              gbuf.at[pl.ds(slot * 16, 16)],
                              gsem.at[slot]).wait()
        lv = lcol_v[pl.ds(j * 16, 16)]
        pltpu.sync_copy(gbuf.at[pl.ds(slot * 16, 16)], acc.at[lv], add=True)
        pltpu.sync_copy(ones_v, accn.at[lv], add=True)

        @pl.when(j + NBUF < ngroups)
        def _():
          fire(j + NBUF)
        return carry

      lax.fori_loop(0, ngroups, step, jnp.int32(0))

      plsc.subcore_barrier()
      for z in range(RPT // ZB):
        r0 = sub * RPT + z * ZB
        pltpu.sync_copy(acc.at[pl.ds(r0, ZB)], sum_r.at[pl.ds(lo + r0, ZB)])
        pltpu.sync_copy(accn.at[pl.ds(r0, ZB)], cnt_r.at[pl.ds(lo + r0, ZB)])
      plsc.subcore_barrier()
      return chunk + 2

    lax.while_loop(lambda ch: ch < nchunks, chunk_body, core)

  ones = jnp.ones((16,), jnp.float32)
  zc = jnp.zeros((ZB, C), jnp.float32)
  z8 = jnp.zeros((ZB,), jnp.float32)
  return sck(src, gidx, bidx, ones, zc, z8)


def _tc_path_update(sums, cnt, xp, WaT, ba, convw, convb, k, R):
  L = xp.shape[0]
  grid = L // R

  def body(sum_ref, cnt_ref, xp_ref, wa_ref, ba_ref, cw_ref, cb_ref, out_ref):
    sc = sum_ref[...] / jnp.maximum(cnt_ref[...], 1.0)
    t = jnp.dot(sc, wa_ref[...], preferred_element_type=jnp.float32)
    h = xp_ref[...] + jnp.maximum(t + ba_ref[...], 0.0)
    pos = lax.broadcasted_iota(jnp.int32, (R, 1), 0) % k
    for d in range(DEPTH):
      hp = jnp.where(pos != 0, pltpu.roll(h, 1, 0), 0.0)
      hn = jnp.where(pos != k - 1, pltpu.roll(h, R - 1, 0), 0.0)
      y = (jnp.dot(hp, cw_ref[d, 0], preferred_element_type=jnp.float32)
           + jnp.dot(h, cw_ref[d, 1], preferred_element_type=jnp.float32)
           + jnp.dot(hn, cw_ref[d, 2], preferred_element_type=jnp.float32)
           + cb_ref[d])
      h = h + jnp.maximum(y, 0.0)
    out_ref[...] = h

  return pl.pallas_call(
      body,
      grid=(grid,),
      in_specs=[
          pl.BlockSpec((R, C), lambda i: (i, 0)),
          pl.BlockSpec((R, 1), lambda i: (i, 0)),
          pl.BlockSpec((R, C), lambda i: (i, 0)),
          pl.BlockSpec((C, C), lambda i: (0, 0)),
          pl.BlockSpec((1, C), lambda i: (0, 0)),
          pl.BlockSpec((DEPTH, 3, C, C), lambda i: (0, 0, 0, 0)),
          pl.BlockSpec((DEPTH, 1, C), lambda i: (0, 0, 0)),
      ],
      out_specs=pl.BlockSpec((R, C), lambda i: (i, 0)),
      out_shape=jax.ShapeDtypeStruct((L, C), jnp.float32),
  )(sums, cnt, xp, WaT, ba, convw, convb)


def _tc_atom_update(x, s3, c3, s5, c5, Wp3T, bp3, Wp5T, bp5, R):
  N = x.shape[0]
  grid = N // R

  def body(x_ref, s3_ref, c3_ref, s5_ref, c5_ref, w3_ref, b3_ref,
           w5_ref, b5_ref, out_ref):
    sc3 = s3_ref[...] / jnp.maximum(c3_ref[...], 1.0)
    sc5 = s5_ref[...] / jnp.maximum(c5_ref[...], 1.0)
    t3 = jnp.dot(sc3, w3_ref[...], preferred_element_type=jnp.float32)
    t5 = jnp.dot(sc5, w5_ref[...], preferred_element_type=jnp.float32)
    out_ref[...] = (x_ref[...]
                    + jnp.maximum(t3 + b3_ref[...], 0.0)
                    + jnp.maximum(t5 + b5_ref[...], 0.0))

  return pl.pallas_call(
      body,
      grid=(grid,),
      in_specs=[
          pl.BlockSpec((R, C), lambda i: (i, 0)),
          pl.BlockSpec((R, C), lambda i: (i, 0)),
          pl.BlockSpec((R, 1), lambda i: (i, 0)),
          pl.BlockSpec((R, C), lambda i: (i, 0)),
          pl.BlockSpec((R, 1), lambda i: (i, 0)),
          pl.BlockSpec((C, C), lambda i: (0, 0)),
          pl.BlockSpec((1, C), lambda i: (0, 0)),
          pl.BlockSpec((C, C), lambda i: (0, 0)),
          pl.BlockSpec((1, C), lambda i: (0, 0)),
      ],
      out_specs=pl.BlockSpec((R, C), lambda i: (i, 0)),
      out_shape=jax.ShapeDtypeStruct((N, C), jnp.float32),
  )(x, s3, c3, s5, c5, Wp3T, bp3, Wp5T, bp5)


def kernel(x, x_paths_3, x_paths_5, row_3, col_3, row_5, col_5,
           Wa_3, ba_3, Wp_3, bp_3, convw_3, convb_3,
           Wa_5, ba_5, Wp_5, bp_5, convw_5, convb_5):
  N = x.shape[0]
  fams = {
      "3": (x_paths_3, row_3, col_3, Wa_3, ba_3, Wp_3, bp_3, convw_3,
            convb_3, 3, 384),
      "5": (x_paths_5, row_5, col_5, Wa_5, ba_5, Wp_5, bp_5, convw_5,
            convb_5, 5, 640),
  }
  new_paths = {}
  mixout = {}
  for name, (xp, row, col, Wa, ba, Wp, bp, cw, cb, k, R) in fams.items():
    L = xp.shape[0]
    sums, cnts = _sc_scatter_sums(x, row, col, L)
    np_ = _tc_path_update(sums, cnts[:, None], xp, Wa.T, ba.reshape(1, C), cw,
                          cb.reshape(DEPTH, 1, C), k, R)
    new_paths[name] = np_
    mixout[name] = _sc_scatter_sums(np_, col, row, N)

  s3, c3 = mixout["3"]
  s5, c5 = mixout["5"]
  x_out = _tc_atom_update(x, s3, c3[:, None], s5, c5[:, None], Wp_3.T, bp_3.reshape(1, C),
                          Wp_5.T, bp_5.reshape(1, C), 800)
  return (x_out, new_paths["3"], new_paths["5"])


# speedup vs baseline: 2.0445x; 1.4416x over previous
import functools

import jax
import jax.numpy as jnp
from jax import lax
from jax.experimental import pallas as pl
from jax.experimental.pallas import tpu as pltpu
from jax.experimental.pallas import tpu_sc as plsc

C = 128
DEPTH = 2
NS = 16
NCORE = 2
CHUNK = 3968
CHUNK_PAD = CHUNK + 16
ZB = 248
NBUF = 3
CNTW = 8


def _sc_scatter_sums(src, gidx, bidx, nbins):
  L = gidx.shape[0]
  nchunks = -(-nbins // CHUNK)
  nbins_pad = nchunks * CHUNK
  EPT = L // NS
  assert L % NS == 0
  NV = -(-EPT // 16)
  CAP = EPT + 16
  RPT = CHUNK // NS

  mesh = plsc.VectorSubcoreMesh(core_axis_name="c", subcore_axis_name="s",
                                num_cores=NCORE, num_subcores=NS)

  @functools.partial(
      pl.kernel,
      out_type=(
          jax.ShapeDtypeStruct((nbins_pad, C), jnp.float32),
          jax.ShapeDtypeStruct((nbins_pad,), jnp.float32),
      ),
      mesh=mesh,
      compiler_params=pltpu.CompilerParams(needs_layout_passes=False),
      scratch_types=(
          pltpu.VMEM((NV * 16,), jnp.int32),
          pltpu.VMEM((NV * 16,), jnp.int32),
          pltpu.VMEM((CAP,), jnp.int32),
          pltpu.VMEM((CAP,), jnp.int32),
          pltpu.VMEM((NBUF * 16, C), jnp.float32),
          pltpu.VMEM((ZB, C), jnp.float32),
          pltpu.VMEM((128,), jnp.float32),
          pltpu.VMEM((16,), jnp.float32),
          pltpu.VMEM_SHARED((CHUNK_PAD, C), jnp.float32),
          pltpu.VMEM_SHARED((CHUNK_PAD,), jnp.float32),
          pltpu.SemaphoreType.DMA((NBUF,)),
      ),
  )
  def sck(src_r, gidx_r, bidx_r, ones_r, zc_r, z8_r, sum_r, cnt_r,
          col_v, row_v, cidx_v, lcol_v, gbuf, zbuf, zcnt, ones_v,
          acc, accn, gsem):
    core = lax.axis_index("c")
    sub = lax.axis_index("s")
    ebase = sub * EPT
    pltpu.sync_copy(bidx_r.at[pl.ds(ebase, EPT)], col_v.at[pl.ds(0, EPT)])
    pltpu.sync_copy(gidx_r.at[pl.ds(ebase, EPT)], row_v.at[pl.ds(0, EPT)])
    pltpu.sync_copy(ones_r, ones_v)
    pltpu.sync_copy(zc_r, zbuf)
    pltpu.sync_copy(z8_r, zcnt)
    lane = lax.broadcasted_iota(jnp.int32, (16,), 0)

    def chunk_body(chunk):
      lo = chunk * CHUNK
      for z in range(RPT // ZB):
        r0 = sub * RPT + z * ZB
        pltpu.sync_copy(zbuf, acc.at[pl.ds(r0, ZB)])
      for u in range((CHUNK // 128 + NS - 1) // NS):
        c0 = (sub + u * NS) * 128
        @pl.when(c0 < CHUNK)
        def _():
          pltpu.sync_copy(zcnt, accn.at[pl.ds(c0, 128)])
      plsc.subcore_barrier()

      def scan_step(v, wptr, tail_mask):
        cv = col_v[pl.ds(v * 16, 16)]
        rv = row_v[pl.ds(v * 16, 16)]
        t = cv - lo
        m = plsc.bitcast(t, jnp.uint32) < jnp.uint32(CHUNK)
        if tail_mask is not None:
          m = m & tail_mask
        mi = jnp.where(m, 1, 0)
        pos = wptr + plsc.cumsum(mi) - 1
        plsc.store_scatter(cidx_v, [pos], rv, mask=m)
        plsc.store_scatter(lcol_v, [pos], t, mask=m)
        return lax.squeeze(lax.slice(pos, (15,), (16,)), (0,)) + 1

      NVF = EPT // 16
      n = lax.fori_loop(0, NVF, lambda v, w: scan_step(v, w, None),
                        jnp.int32(0), unroll=4)
      if EPT % 16:
        n = scan_step(jnp.int32(NVF), n, lane < (EPT % 16))
      plsc.store_scatter(cidx_v, [n + lane], jnp.zeros((16,), jnp.int32))
      plsc.store_scatter(lcol_v, [n + lane], jnp.full((16,), CHUNK, jnp.int32))
      ngroups = (n + 15) // 16

      def fire(j):
        slot = lax.rem(j, NBUF)
        idxv = cidx_v[pl.ds(j * 16, 16)]
        pltpu.async_copy(src_r.at[idxv], gbuf.at[pl.ds(slot * 16, 16)],
                         gsem.at[slot])

      for b in range(NBUF):
        @pl.when(b < ngroups)
        def _():
          fire(jnp.int32(b))

      def step(j, carry):
        slot = lax.rem(j, NBUF)
        idxv = cidx_v[pl.ds(j * 16, 16)]
        pltpu.make_async_copy(src_r.at[idxv],
                              gbuf.at[pl.ds(slot * 16, 16)],
                              gsem.at[slot]).wait()
        lv = lcol_v[pl.ds(j * 16, 16)]
        pltpu.sync_copy(gbuf.at[pl.ds(slot * 16, 16)], acc.at[lv], add=True)
        pltpu.sync_copy(ones_v, accn.at[lv], add=True)

        @pl.when(j + NBUF < ngroups)
        def _():
          fire(j + NBUF)
        return carry

      lax.fori_loop(0, ngroups, step, jnp.int32(0))

      plsc.subcore_barrier()
      for z in range(RPT // ZB):
        r0 = sub * RPT + z * ZB
        pltpu.sync_copy(acc.at[pl.ds(r0, ZB)], sum_r.at[pl.ds(lo + r0, ZB)])
      for u in range((CHUNK // 128 + NS - 1) // NS):
        c0 = (sub + u * NS) * 128
        @pl.when(c0 < CHUNK)
        def _():
          pltpu.sync_copy(accn.at[pl.ds(c0, 128)], cnt_r.at[pl.ds(lo + c0, 128)])
      plsc.subcore_barrier()
      return chunk + 2

    lax.while_loop(lambda ch: ch < nchunks, chunk_body, core)

  ones = jnp.ones((16,), jnp.float32)
  zc = jnp.zeros((ZB, C), jnp.float32)
  z8 = jnp.zeros((128,), jnp.float32)
  return sck(src, gidx, bidx, ones, zc, z8)


def _tc_path_update(sums, cnt, xp, WaT, ba, convw, convb, k, R):
  L = xp.shape[0]
  grid = L // R

  def body(sum_ref, cnt_ref, xp_ref, wa_ref, ba_ref, cw_ref, cb_ref, out_ref):
    sc = sum_ref[...] / jnp.maximum(cnt_ref[...], 1.0)
    t = jnp.dot(sc, wa_ref[...], preferred_element_type=jnp.float32)
    h = xp_ref[...] + jnp.maximum(t + ba_ref[...], 0.0)
    pos = lax.broadcasted_iota(jnp.int32, (R, 1), 0) % k
    for d in range(DEPTH):
      hp = jnp.where(pos != 0, pltpu.roll(h, 1, 0), 0.0)
      hn = jnp.where(pos != k - 1, pltpu.roll(h, R - 1, 0), 0.0)
      y = (jnp.dot(hp, cw_ref[d, 0], preferred_element_type=jnp.float32)
           + jnp.dot(h, cw_ref[d, 1], preferred_element_type=jnp.float32)
           + jnp.dot(hn, cw_ref[d, 2], preferred_element_type=jnp.float32)
           + cb_ref[d])
      h = h + jnp.maximum(y, 0.0)
    out_ref[...] = h

  return pl.pallas_call(
      body,
      grid=(grid,),
      in_specs=[
          pl.BlockSpec((R, C), lambda i: (i, 0)),
          pl.BlockSpec((R, 1), lambda i: (i, 0)),
          pl.BlockSpec((R, C), lambda i: (i, 0)),
          pl.BlockSpec((C, C), lambda i: (0, 0)),
          pl.BlockSpec((1, C), lambda i: (0, 0)),
          pl.BlockSpec((DEPTH, 3, C, C), lambda i: (0, 0, 0, 0)),
          pl.BlockSpec((DEPTH, 1, C), lambda i: (0, 0, 0)),
      ],
      out_specs=pl.BlockSpec((R, C), lambda i: (i, 0)),
      out_shape=jax.ShapeDtypeStruct((L, C), jnp.float32),
  )(sums, cnt, xp, WaT, ba, convw, convb)


def _tc_atom_update(x, s3, c3, s5, c5, Wp3T, bp3, Wp5T, bp5, R):
  N = x.shape[0]
  grid = N // R

  def body(x_ref, s3_ref, c3_ref, s5_ref, c5_ref, w3_ref, b3_ref,
           w5_ref, b5_ref, out_ref):
    sc3 = s3_ref[...] / jnp.maximum(c3_ref[...], 1.0)
    sc5 = s5_ref[...] / jnp.maximum(c5_ref[...], 1.0)
    t3 = jnp.dot(sc3, w3_ref[...], preferred_element_type=jnp.float32)
    t5 = jnp.dot(sc5, w5_ref[...], preferred_element_type=jnp.float32)
    out_ref[...] = (x_ref[...]
                    + jnp.maximum(t3 + b3_ref[...], 0.0)
                    + jnp.maximum(t5 + b5_ref[...], 0.0))

  return pl.pallas_call(
      body,
      grid=(grid,),
      in_specs=[
          pl.BlockSpec((R, C), lambda i: (i, 0)),
          pl.BlockSpec((R, C), lambda i: (i, 0)),
          pl.BlockSpec((R, 1), lambda i: (i, 0)),
          pl.BlockSpec((R, C), lambda i: (i, 0)),
          pl.BlockSpec((R, 1), lambda i: (i, 0)),
          pl.BlockSpec((C, C), lambda i: (0, 0)),
          pl.BlockSpec((1, C), lambda i: (0, 0)),
          pl.BlockSpec((C, C), lambda i: (0, 0)),
          pl.BlockSpec((1, C), lambda i: (0, 0)),
      ],
      out_specs=pl.BlockSpec((R, C), lambda i: (i, 0)),
      out_shape=jax.ShapeDtypeStruct((N, C), jnp.float32),
  )(x, s3, c3, s5, c5, Wp3T, bp3, Wp5T, bp5)


def kernel(x, x_paths_3, x_paths_5, row_3, col_3, row_5, col_5,
           Wa_3, ba_3, Wp_3, bp_3, convw_3, convb_3,
           Wa_5, ba_5, Wp_5, bp_5, convw_5, convb_5):
  N = x.shape[0]
  fams = {
      "3": (x_paths_3, row_3, col_3, Wa_3, ba_3, Wp_3, bp_3, convw_3,
            convb_3, 3, 384),
      "5": (x_paths_5, row_5, col_5, Wa_5, ba_5, Wp_5, bp_5, convw_5,
            convb_5, 5, 640),
  }
  new_paths = {}
  mixout = {}
  for name, (xp, row, col, Wa, ba, Wp, bp, cw, cb, k, R) in fams.items():
    L = xp.shape[0]
    sums, cnts = _sc_scatter_sums(x, row, col, L)
    np_ = _tc_path_update(sums, cnts[:, None], xp, Wa.T, ba.reshape(1, C), cw,
                          cb.reshape(DEPTH, 1, C), k, R)
    new_paths[name] = np_
    mixout[name] = _sc_scatter_sums(np_, col, row, N)

  s3, c3 = mixout["3"]
  s5, c5 = mixout["5"]
  x_out = _tc_atom_update(x, s3, c3[:, None], s5, c5[:, None], Wp_3.T, bp_3.reshape(1, C),
                          Wp_5.T, bp_5.reshape(1, C), 800)
  return (x_out, new_paths["3"], new_paths["5"])
